# baseline (speedup 1.0000x reference)
import functools

import jax
import jax.numpy as jnp
from jax import lax
from jax.experimental import pallas as pl
from jax.experimental.pallas import tpu as pltpu
from jax.experimental.pallas import tpu_sc as plsc

NUM_CORES = 2
NUM_SUBCORES = 16
NUM_WORKERS = NUM_CORES * NUM_SUBCORES
LANES = 16

BATCH = 16384
DIM = 64
SUB = 8
B_PER_W = BATCH // NUM_WORKERS
CHUNK = 16
N_CHUNKS = B_PER_W // CHUNK
BUF_ROWS = CHUNK * SUB


def _mf_body(user_hbm, item_hbm, uemb_hbm, iemb_hbm, ubias_hbm, ibias_hbm,
             gbias_hbm, out_hbm,
             uidx_v, iidx_v, ur0, ir0, ur1, ir1, ubb, ibb,
             out_v, part_v, gb_v, sem0, sem1, semb):
    wid = lax.axis_index("s") * NUM_CORES + lax.axis_index("c")
    base = wid * B_PER_W

    pltpu.sync_copy(user_hbm.at[pl.ds(base, B_PER_W)], uidx_v)
    pltpu.sync_copy(item_hbm.at[pl.ds(base, B_PER_W)], iidx_v)
    pltpu.sync_copy(gbias_hbm, gb_v.at[pl.ds(0, 1)])

    row_bufs = [(ur0, ir0, sem0), (ur1, ir1, sem1)]

    def fire_rows(c, parity):
        urb, irb, sem = row_bufs[parity]
        u_vec = uidx_v[pl.ds(c * CHUNK, CHUNK)]
        i_vec = iidx_v[pl.ds(c * CHUNK, CHUNK)]
        ublk = (u_vec // SUB) * SUB
        iblk = (i_vec // SUB) * SUB
        for r in range(CHUNK):
            ub = pl.multiple_of(ublk[r], SUB)
            ib = pl.multiple_of(iblk[r], SUB)
            pltpu.async_copy(uemb_hbm.at[pl.ds(ub, SUB), :],
                             urb.at[pl.ds(r * SUB, SUB), :], sem)
            pltpu.async_copy(iemb_hbm.at[pl.ds(ib, SUB), :],
                             irb.at[pl.ds(r * SUB, SUB), :], sem)

    def fire_bias(c):
        u_vec = uidx_v[pl.ds(c * CHUNK, CHUNK)]
        i_vec = iidx_v[pl.ds(c * CHUNK, CHUNK)]
        ublk = (u_vec // SUB) * SUB
        iblk = (i_vec // SUB) * SUB
        for r in range(CHUNK):
            ub = pl.multiple_of(ublk[r], SUB)
            ib = pl.multiple_of(iblk[r], SUB)
            pltpu.async_copy(ubias_hbm.at[pl.ds(ub, SUB), :],
                             ubb.at[pl.ds(r * SUB, SUB), :], semb)
            pltpu.async_copy(ibias_hbm.at[pl.ds(ib, SUB), :],
                             ibb.at[pl.ds(r * SUB, SUB), :], semb)

    def drain_rows(parity):
        urb, irb, sem = row_bufs[parity]
        pltpu.make_async_copy(uemb_hbm.at[pl.ds(0, BUF_ROWS), :], urb,
                              sem).wait()
        pltpu.make_async_copy(iemb_hbm.at[pl.ds(0, BUF_ROWS), :], irb,
                              sem).wait()

    def drain_bias():
        pltpu.make_async_copy(ubias_hbm.at[pl.ds(0, BUF_ROWS), :], ubb,
                              semb).wait()
        pltpu.make_async_copy(ibias_hbm.at[pl.ds(0, BUF_ROWS), :], ibb,
                              semb).wait()

    lane_iota = lax.iota(jnp.int32, LANES)
    zero_idx = lane_iota * 0

    def compute_chunk(c, parity):
        urb, irb, _ = row_bufs[parity]
        gb = gb_v[...][0]
        u_vec = uidx_v[pl.ds(c * CHUNK, CHUNK)]
        i_vec = iidx_v[pl.ds(c * CHUNK, CHUNK)]
        u_sub = u_vec % SUB
        i_sub = i_vec % SUB
        for r in range(CHUNK):
            urow = urb.at[r * SUB + u_sub[r]]
            irow = irb.at[r * SUB + i_sub[r]]
            s = urow[pl.ds(0, LANES)] * irow[pl.ds(0, LANES)]
            for cc in range(1, DIM // LANES):
                s = s + (urow[pl.ds(cc * LANES, LANES)]
                         * irow[pl.ds(cc * LANES, LANES)])
            part_v[pl.ds(r * LANES, LANES)] = s
        slot = lane_iota * SUB
        bu = plsc.load_gather(ubb, [slot + u_sub, zero_idx])
        bi = plsc.load_gather(ibb, [slot + i_sub, zero_idx])
        acc = bu + bi + gb
        for cc in range(LANES):
            acc = acc + plsc.load_gather(part_v, [lane_iota * LANES + cc])
        out_v[pl.ds(c * CHUNK, LANES)] = acc

    fire_bias(0)
    fire_rows(0, 0)
    fire_rows(1, 1)
    drain_bias()

    def step(c, parity, last):
        drain_rows(parity)
        compute_chunk(c, parity)
        if not last:
            fire_bias(c + 1)
            fire_rows(c + 2, parity)
            drain_bias()

    def pipeline_body(k, carry):
        c0 = 2 * k
        step(c0, 0, False)
        step(c0 + 1, 1, False)
        return carry

    lax.fori_loop(0, N_CHUNKS // 2 - 1, pipeline_body, 0)
    drain_rows(0)
    compute_chunk(N_CHUNKS - 2, 0)
    fire_bias(N_CHUNKS - 1)
    drain_bias()
    step(N_CHUNKS - 1, 1, True)

    pltpu.sync_copy(out_v, out_hbm.at[pl.ds(base, B_PER_W)])


_mf_kernel = functools.partial(
    pl.kernel,
    out_type=jax.ShapeDtypeStruct((BATCH,), jnp.float32),
    mesh=plsc.VectorSubcoreMesh(core_axis_name="c", subcore_axis_name="s",
                                num_cores=NUM_CORES,
                                num_subcores=NUM_SUBCORES),
    scratch_types=[
        pltpu.VMEM((B_PER_W,), jnp.int32),
        pltpu.VMEM((B_PER_W,), jnp.int32),
        pltpu.VMEM((BUF_ROWS, DIM), jnp.float32),
        pltpu.VMEM((BUF_ROWS, DIM), jnp.float32),
        pltpu.VMEM((BUF_ROWS, DIM), jnp.float32),
        pltpu.VMEM((BUF_ROWS, DIM), jnp.float32),
        pltpu.VMEM((BUF_ROWS, 1), jnp.float32),
        pltpu.VMEM((BUF_ROWS, 1), jnp.float32),
        pltpu.VMEM((B_PER_W,), jnp.float32),
        pltpu.VMEM((LANES * LANES,), jnp.float32),
        pltpu.VMEM((LANES,), jnp.float32),
        pltpu.SemaphoreType.DMA,
        pltpu.SemaphoreType.DMA,
        pltpu.SemaphoreType.DMA,
    ],
    compiler_params=pltpu.CompilerParams(needs_layout_passes=False,
                                         use_tc_tiling_on_sc=True),
)(_mf_body)


@jax.jit
def kernel(user, item, user_emb, item_emb, user_bias, item_bias, global_bias):
    user = user.astype(jnp.int32)
    item = item.astype(jnp.int32)
    return _mf_kernel(user, item, user_emb, item_emb,
                      user_bias, item_bias, global_bias)


# speedup vs baseline: 1.4703x; 1.4703x over previous
import functools

import jax
import jax.numpy as jnp
from jax import lax
from jax.experimental import pallas as pl
from jax.experimental.pallas import tpu as pltpu
from jax.experimental.pallas import tpu_sc as plsc

NUM_CORES = 2
NUM_SUBCORES = 16
NUM_WORKERS = NUM_CORES * NUM_SUBCORES
LANES = 16

BATCH = 16384
DIM = 64
SUB = 8
B_PER_W = BATCH // NUM_WORKERS
CHUNK = 16
N_CHUNKS = B_PER_W // CHUNK
BUF_ROWS = CHUNK * SUB


def _mf_body(user_hbm, item_hbm, uemb_hbm, iemb_hbm, ubias_hbm, ibias_hbm,
             gbias_hbm, out_hbm,
             uidx_v, iidx_v, ur0, ir0, ur1, ir1, ubb, ibb,
             out_v, part_v, gb_v, sem0, sem1, semb):
    wid = lax.axis_index("s") * NUM_CORES + lax.axis_index("c")
    base = wid * B_PER_W

    pltpu.sync_copy(user_hbm.at[pl.ds(base, B_PER_W)], uidx_v)
    pltpu.sync_copy(item_hbm.at[pl.ds(base, B_PER_W)], iidx_v)
    pltpu.sync_copy(gbias_hbm, gb_v.at[pl.ds(0, 1)])

    row_bufs = [(ur0, ir0, sem0), (ur1, ir1, sem1)]

    def fire_rows(c, parity):
        urb, irb, sem = row_bufs[parity]
        u_vec = uidx_v[pl.ds(c * CHUNK, CHUNK)]
        i_vec = iidx_v[pl.ds(c * CHUNK, CHUNK)]
        ublk = (u_vec // SUB) * SUB
        iblk = (i_vec // SUB) * SUB
        for r in range(CHUNK):
            ub = pl.multiple_of(ublk[r], SUB)
            ib = pl.multiple_of(iblk[r], SUB)
            pltpu.async_copy(uemb_hbm.at[pl.ds(ub, SUB), :],
                             urb.at[pl.ds(r * SUB, SUB), :], sem)
            pltpu.async_copy(iemb_hbm.at[pl.ds(ib, SUB), :],
                             irb.at[pl.ds(r * SUB, SUB), :], sem)

    def fire_bias(g, carry):
        u_vec = uidx_v[pl.ds(g * LANES, LANES)]
        i_vec = iidx_v[pl.ds(g * LANES, LANES)]
        ublk = (u_vec // SUB) * SUB
        iblk = (i_vec // SUB) * SUB
        for r in range(LANES):
            j = g * LANES + r
            ub = pl.multiple_of(ublk[r], SUB)
            ib = pl.multiple_of(iblk[r], SUB)
            pltpu.async_copy(ubias_hbm.at[pl.ds(ub, SUB)],
                             ubb.at[pl.ds(j * SUB, SUB)], semb)
            pltpu.async_copy(ibias_hbm.at[pl.ds(ib, SUB)],
                             ibb.at[pl.ds(j * SUB, SUB)], semb)
        return carry

    def drain_rows(parity):
        urb, irb, sem = row_bufs[parity]
        pltpu.make_async_copy(uemb_hbm.at[pl.ds(0, BUF_ROWS), :], urb,
                              sem).wait()
        pltpu.make_async_copy(iemb_hbm.at[pl.ds(0, BUF_ROWS), :], irb,
                              sem).wait()

    def drain_bias():
        pltpu.make_async_copy(out_hbm.at[pl.ds(0, B_PER_W * SUB)], ubb,
                              semb).wait()
        pltpu.make_async_copy(out_hbm.at[pl.ds(0, B_PER_W * SUB)], ibb,
                              semb).wait()

    lane_iota = lax.iota(jnp.int32, LANES)
    zero_idx = lane_iota * 0

    def compute_chunk(c, parity):
        urb, irb, _ = row_bufs[parity]
        gb = gb_v[...][0]
        u_vec = uidx_v[pl.ds(c * CHUNK, CHUNK)]
        i_vec = iidx_v[pl.ds(c * CHUNK, CHUNK)]
        u_sub = u_vec % SUB
        i_sub = i_vec % SUB
        for r in range(CHUNK):
            urow = urb.at[r * SUB + u_sub[r]]
            irow = irb.at[r * SUB + i_sub[r]]
            s = urow[pl.ds(0, LANES)] * irow[pl.ds(0, LANES)]
            for cc in range(1, DIM // LANES):
                s = s + (urow[pl.ds(cc * LANES, LANES)]
                         * irow[pl.ds(cc * LANES, LANES)])
            part_v[pl.ds(r * LANES, LANES)] = s
        slot = (lane_iota + c * CHUNK) * SUB
        bu = plsc.load_gather(ubb, [slot + u_sub])
        bi = plsc.load_gather(ibb, [slot + i_sub])
        acc = bu + bi + gb
        for cc in range(LANES):
            acc = acc + plsc.load_gather(part_v, [lane_iota * LANES + cc])
        out_v[pl.ds(c * CHUNK, LANES)] = acc

    lax.fori_loop(0, B_PER_W // LANES, fire_bias, 0)
    fire_rows(0, 0)
    fire_rows(1, 1)
    drain_bias()

    def pipeline_body(k, carry):
        c0 = 2 * k
        drain_rows(0)
        compute_chunk(c0, 0)
        fire_rows(c0 + 2, 0)
        drain_rows(1)
        compute_chunk(c0 + 1, 1)
        fire_rows(c0 + 3, 1)
        return carry

    lax.fori_loop(0, N_CHUNKS // 2 - 1, pipeline_body, 0)
    drain_rows(0)
    compute_chunk(N_CHUNKS - 2, 0)
    drain_rows(1)
    compute_chunk(N_CHUNKS - 1, 1)

    pltpu.sync_copy(out_v, out_hbm.at[pl.ds(base, B_PER_W)])


_mf_kernel = functools.partial(
    pl.kernel,
    out_type=jax.ShapeDtypeStruct((BATCH,), jnp.float32),
    mesh=plsc.VectorSubcoreMesh(core_axis_name="c", subcore_axis_name="s",
                                num_cores=NUM_CORES,
                                num_subcores=NUM_SUBCORES),
    scratch_types=[
        pltpu.VMEM((B_PER_W,), jnp.int32),
        pltpu.VMEM((B_PER_W,), jnp.int32),
        pltpu.VMEM((BUF_ROWS, DIM), jnp.float32),
        pltpu.VMEM((BUF_ROWS, DIM), jnp.float32),
        pltpu.VMEM((BUF_ROWS, DIM), jnp.float32),
        pltpu.VMEM((BUF_ROWS, DIM), jnp.float32),
        pltpu.VMEM((B_PER_W * SUB,), jnp.float32),
        pltpu.VMEM((B_PER_W * SUB,), jnp.float32),
        pltpu.VMEM((B_PER_W,), jnp.float32),
        pltpu.VMEM((LANES * LANES,), jnp.float32),
        pltpu.VMEM((LANES,), jnp.float32),
        pltpu.SemaphoreType.DMA,
        pltpu.SemaphoreType.DMA,
        pltpu.SemaphoreType.DMA,
    ],
    compiler_params=pltpu.CompilerParams(needs_layout_passes=False,
                                         use_tc_tiling_on_sc=True),
)(_mf_body)


@jax.jit
def kernel(user, item, user_emb, item_emb, user_bias, item_bias, global_bias):
    user = user.astype(jnp.int32)
    item = item.astype(jnp.int32)
    return _mf_kernel(user, item, user_emb, item_emb,
                      user_bias.reshape(-1), item_bias.reshape(-1),
                      global_bias)


# speedup vs baseline: 1.5780x; 1.0733x over previous
import functools

import jax
import jax.numpy as jnp
from jax import lax
from jax.experimental import pallas as pl
from jax.experimental.pallas import tpu as pltpu
from jax.experimental.pallas import tpu_sc as plsc

NUM_CORES = 2
NUM_SUBCORES = 16
NUM_WORKERS = NUM_CORES * NUM_SUBCORES
LANES = 16

BATCH = 16384
DIM = 64
SUB = 8
B_PER_W = BATCH // NUM_WORKERS
CHUNK = 16
N_CHUNKS = B_PER_W // CHUNK
BUF_ROWS = CHUNK * SUB

_MESH = plsc.VectorSubcoreMesh(core_axis_name="c", subcore_axis_name="s",
                               num_cores=NUM_CORES,
                               num_subcores=NUM_SUBCORES)


def _dot_body(user_hbm, item_hbm, uemb_hbm, iemb_hbm, out_hbm,
              uidx_v, iidx_v, ur0, ir0, ur1, ir1, out_v, part_v,
              sem0, sem1):
    wid = lax.axis_index("s") * NUM_CORES + lax.axis_index("c")
    base = wid * B_PER_W

    pltpu.sync_copy(user_hbm.at[pl.ds(base, B_PER_W)], uidx_v)
    pltpu.sync_copy(item_hbm.at[pl.ds(base, B_PER_W)], iidx_v)

    row_bufs = [(ur0, ir0, sem0), (ur1, ir1, sem1)]

    def fire_rows(c, parity):
        urb, irb, sem = row_bufs[parity]
        u_vec = uidx_v[pl.ds(c * CHUNK, CHUNK)]
        i_vec = iidx_v[pl.ds(c * CHUNK, CHUNK)]
        ublk = (u_vec // SUB) * SUB
        iblk = (i_vec // SUB) * SUB
        for r in range(CHUNK):
            ub = pl.multiple_of(ublk[r], SUB)
            ib = pl.multiple_of(iblk[r], SUB)
            pltpu.async_copy(uemb_hbm.at[pl.ds(ub, SUB), :],
                             urb.at[pl.ds(r * SUB, SUB), :], sem)
            pltpu.async_copy(iemb_hbm.at[pl.ds(ib, SUB), :],
                             irb.at[pl.ds(r * SUB, SUB), :], sem)

    def drain_rows(parity):
        urb, irb, sem = row_bufs[parity]
        pltpu.make_async_copy(uemb_hbm.at[pl.ds(0, BUF_ROWS), :], urb,
                              sem).wait()
        pltpu.make_async_copy(iemb_hbm.at[pl.ds(0, BUF_ROWS), :], irb,
                              sem).wait()

    lane_iota = lax.iota(jnp.int32, LANES)

    def compute_chunk(c, parity):
        urb, irb, _ = row_bufs[parity]
        u_vec = uidx_v[pl.ds(c * CHUNK, CHUNK)]
        i_vec = iidx_v[pl.ds(c * CHUNK, CHUNK)]
        u_sub = u_vec % SUB
        i_sub = i_vec % SUB
        for r in range(CHUNK):
            urow = urb.at[r * SUB + u_sub[r]]
            irow = irb.at[r * SUB + i_sub[r]]
            s = urow[pl.ds(0, LANES)] * irow[pl.ds(0, LANES)]
            for cc in range(1, DIM // LANES):
                s = s + (urow[pl.ds(cc * LANES, LANES)]
                         * irow[pl.ds(cc * LANES, LANES)])
            part_v[pl.ds(r * LANES, LANES)] = s
        acc = plsc.load_gather(part_v, [lane_iota * LANES])
        for cc in range(1, LANES):
            acc = acc + plsc.load_gather(part_v, [lane_iota * LANES + cc])
        out_v[pl.ds(c * CHUNK, LANES)] = acc

    fire_rows(0, 0)
    fire_rows(1, 1)

    def pipeline_body(k, carry):
        c0 = 2 * k
        drain_rows(0)
        compute_chunk(c0, 0)
        fire_rows(c0 + 2, 0)
        drain_rows(1)
        compute_chunk(c0 + 1, 1)
        fire_rows(c0 + 3, 1)
        return carry

    lax.fori_loop(0, N_CHUNKS // 2 - 1, pipeline_body, 0)
    drain_rows(0)
    compute_chunk(N_CHUNKS - 2, 0)
    drain_rows(1)
    compute_chunk(N_CHUNKS - 1, 1)

    pltpu.sync_copy(out_v, out_hbm.at[pl.ds(base, B_PER_W)])


def _bias_body(user_hbm, item_hbm, ubias_hbm, ibias_hbm, gbias_hbm,
               partial_hbm, out_hbm,
               uidx_v, iidx_v, ubb, ibb, pin_v, out_v, gb_v, semb):
    wid = lax.axis_index("s") * NUM_CORES + lax.axis_index("c")
    base = wid * B_PER_W

    pltpu.sync_copy(user_hbm.at[pl.ds(base, B_PER_W)], uidx_v)
    pltpu.sync_copy(item_hbm.at[pl.ds(base, B_PER_W)], iidx_v)
    pltpu.sync_copy(gbias_hbm, gb_v.at[pl.ds(0, 1)])

    def fire_bias(g, carry):
        u_vec = uidx_v[pl.ds(g * LANES, LANES)]
        i_vec = iidx_v[pl.ds(g * LANES, LANES)]
        ublk = (u_vec // SUB) * SUB
        iblk = (i_vec // SUB) * SUB
        for r in range(LANES):
            j = g * LANES + r
            ub = pl.multiple_of(ublk[r], SUB)
            ib = pl.multiple_of(iblk[r], SUB)
            pltpu.async_copy(ubias_hbm.at[pl.ds(ub, SUB)],
                             ubb.at[pl.ds(j * SUB, SUB)], semb)
            pltpu.async_copy(ibias_hbm.at[pl.ds(ib, SUB)],
                             ibb.at[pl.ds(j * SUB, SUB)], semb)
        return carry

    lax.fori_loop(0, B_PER_W // LANES, fire_bias, 0)
    pltpu.sync_copy(partial_hbm.at[pl.ds(base, B_PER_W)], pin_v)
    pltpu.make_async_copy(out_hbm.at[pl.ds(0, B_PER_W * SUB)], ubb,
                          semb).wait()
    pltpu.make_async_copy(out_hbm.at[pl.ds(0, B_PER_W * SUB)], ibb,
                          semb).wait()

    lane_iota = lax.iota(jnp.int32, LANES)
    gb = gb_v[...][0]

    def add_group(g, carry):
        u_vec = uidx_v[pl.ds(g * LANES, LANES)]
        i_vec = iidx_v[pl.ds(g * LANES, LANES)]
        slot = (lane_iota + g * LANES) * SUB
        bu = plsc.load_gather(ubb, [slot + u_vec % SUB])
        bi = plsc.load_gather(ibb, [slot + i_vec % SUB])
        acc = pin_v[pl.ds(g * LANES, LANES)] + bu + bi + gb
        out_v[pl.ds(g * LANES, LANES)] = acc
        return carry

    lax.fori_loop(0, B_PER_W // LANES, add_group, 0)

    pltpu.sync_copy(out_v, out_hbm.at[pl.ds(base, B_PER_W)])


_dot_kernel = functools.partial(
    pl.kernel,
    out_type=jax.ShapeDtypeStruct((BATCH,), jnp.float32),
    mesh=_MESH,
    scratch_types=[
        pltpu.VMEM((B_PER_W,), jnp.int32),
        pltpu.VMEM((B_PER_W,), jnp.int32),
        pltpu.VMEM((BUF_ROWS, DIM), jnp.float32),
        pltpu.VMEM((BUF_ROWS, DIM), jnp.float32),
        pltpu.VMEM((BUF_ROWS, DIM), jnp.float32),
        pltpu.VMEM((BUF_ROWS, DIM), jnp.float32),
        pltpu.VMEM((B_PER_W,), jnp.float32),
        pltpu.VMEM((LANES * LANES,), jnp.float32),
        pltpu.SemaphoreType.DMA,
        pltpu.SemaphoreType.DMA,
    ],
    compiler_params=pltpu.CompilerParams(needs_layout_passes=False,
                                         use_tc_tiling_on_sc=True),
)(_dot_body)


_bias_kernel = functools.partial(
    pl.kernel,
    out_type=jax.ShapeDtypeStruct((BATCH,), jnp.float32),
    mesh=_MESH,
    scratch_types=[
        pltpu.VMEM((B_PER_W,), jnp.int32),
        pltpu.VMEM((B_PER_W,), jnp.int32),
        pltpu.VMEM((B_PER_W * SUB,), jnp.float32),
        pltpu.VMEM((B_PER_W * SUB,), jnp.float32),
        pltpu.VMEM((B_PER_W,), jnp.float32),
        pltpu.VMEM((B_PER_W,), jnp.float32),
        pltpu.VMEM((LANES,), jnp.float32),
        pltpu.SemaphoreType.DMA,
    ],
    compiler_params=pltpu.CompilerParams(needs_layout_passes=False,
                                         use_tc_tiling_on_sc=True),
)(_bias_body)


@jax.jit
def kernel(user, item, user_emb, item_emb, user_bias, item_bias, global_bias):
    user = user.astype(jnp.int32)
    item = item.astype(jnp.int32)
    partial = _dot_kernel(user, item, user_emb, item_emb)
    return _bias_kernel(user, item, user_bias.reshape(-1),
                        item_bias.reshape(-1), global_bias, partial)
